# baseline (speedup 1.0000x reference)
import functools

import jax
import jax.numpy as jnp
from jax import lax
from jax.experimental import pallas as pl
from jax.experimental.pallas import tpu as pltpu
from jax.experimental.pallas import tpu_sc as plsc

_D_MODEL = 4096
_N_EXPERTS = 80
_TOP_K = 8
_N_TOK = 8192
_N_CHUNK = 4
_CHUNK = _N_TOK // _N_CHUNK
_N_WORKERS = 32
_TPT = _CHUNK // _N_WORKERS
_TB = 256
_WPB = _TB // _TPT
_BPB = (_N_TOK // 2) // _TB

_NEG_INF = float("-inf")


def _dense_body(x_ref, w_ref, b_ref, probs_ref, logt_ref):
    xb = x_ref[...]
    w = w_ref[...]
    logits = lax.dot_general(
        xb, w, dimension_numbers=(((1,), (1,)), ((), ())),
        preferred_element_type=jnp.float32,
    )
    logits = logits + b_ref[...]

    m = jnp.max(logits, axis=1, keepdims=True)
    e = jnp.exp(logits - m)
    probs_ref[...] = e / jnp.sum(e, axis=1, keepdims=True)

    lt = logits.T
    logt_ref[...] = jnp.concatenate(
        [lt[:, w * _TPT:(w + 1) * _TPT].reshape(1, _N_EXPERTS, _TPT)
         for w in range(_WPB)], axis=0)


def _run_dense(x2d, W, bias, chunk):
    grid = (_CHUNK // _TB,)
    base = chunk * (_CHUNK // _TB)
    return pl.pallas_call(
        _dense_body,
        grid=grid,
        in_specs=[
            pl.BlockSpec((_TB, _D_MODEL), lambda i: (base + i, 0)),
            pl.BlockSpec((_N_EXPERTS, _D_MODEL), lambda i: (0, 0)),
            pl.BlockSpec((1, _N_EXPERTS), lambda i: (0, 0)),
        ],
        out_specs=[
            pl.BlockSpec((_TB, _N_EXPERTS), lambda i: (i, 0)),
            pl.BlockSpec((_WPB, _N_EXPERTS, _TPT), lambda i: (i, 0, 0)),
        ],
        out_shape=[
            jax.ShapeDtypeStruct((_CHUNK, _N_EXPERTS), jnp.float32),
            jax.ShapeDtypeStruct((_N_WORKERS, _N_EXPERTS, _TPT), jnp.float32),
        ],
        name=f"router_dense_c{chunk}",
    )(x2d, W, bias)


def _topk_body(logt_hbm, wts_hbm, idx_hbm, buf, wv, iv):
    c = lax.axis_index("c")
    s = lax.axis_index("s")
    wid = s * 2 + c

    pltpu.sync_copy(logt_hbm.at[wid], buf)

    def group(g, carry):
        base = g * 16
        t_v = [jnp.full((16,), _NEG_INF, jnp.float32) for _ in range(_TOP_K)]
        t_i = [jnp.zeros((16,), jnp.int32) for _ in range(_TOP_K)]
        for e in range(_N_EXPERTS):
            v = buf[e, pl.ds(base, 16)]
            ei = jnp.full((16,), e, jnp.int32)
            b = [v > t_v[j] for j in range(_TOP_K)]
            nv = [jnp.where(b[0], v, t_v[0])]
            ni = [jnp.where(b[0], ei, t_i[0])]
            for j in range(1, _TOP_K):
                cv = jnp.where(b[j - 1], t_v[j - 1], v)
                ci = jnp.where(b[j - 1], t_i[j - 1], ei)
                nv.append(jnp.where(b[j], cv, t_v[j]))
                ni.append(jnp.where(b[j], ci, t_i[j]))
            t_v, t_i = nv, ni
        es = [jnp.exp(t_v[j] - t_v[0]) for j in range(_TOP_K)]
        tot = es[0]
        for j in range(1, _TOP_K):
            tot = tot + es[j]
        for j in range(_TOP_K):
            wv[j, pl.ds(base, 16)] = es[j] / tot
            iv[j, pl.ds(base, 16)] = t_i[j]
        return carry

    lax.fori_loop(0, _TPT // 16, group, 0)

    pltpu.sync_copy(wv, wts_hbm.at[wid])
    pltpu.sync_copy(iv, idx_hbm.at[wid])


def _run_topk(logt):
    mesh = plsc.VectorSubcoreMesh(core_axis_name="c", subcore_axis_name="s")
    f = pl.kernel(
        _topk_body,
        out_type=[
            jax.ShapeDtypeStruct((_N_WORKERS, _TOP_K, _TPT), jnp.float32),
            jax.ShapeDtypeStruct((_N_WORKERS, _TOP_K, _TPT), jnp.int32),
        ],
        mesh=mesh,
        scratch_types=[
            pltpu.VMEM((_N_EXPERTS, _TPT), jnp.float32),
            pltpu.VMEM((_TOP_K, _TPT), jnp.float32),
            pltpu.VMEM((_TOP_K, _TPT), jnp.int32),
        ],
    )
    return f(logt)


@jax.jit
def kernel(x, W, expert_bias):
    b, seq, d = x.shape
    x2d = x.reshape(b * seq, d)
    bias2d = expert_bias.reshape(1, _N_EXPERTS)
    probs_cs, wts_cs, idx_cs = [], [], []
    for c in range(_N_CHUNK):
        probs_c, logt_c = _run_dense(x2d, W, bias2d, c)
        wts3, idx3 = _run_topk(logt_c)
        probs_cs.append(probs_c)
        wts_cs.append(wts3.transpose(0, 2, 1).reshape(_CHUNK, _TOP_K))
        idx_cs.append(idx3.transpose(0, 2, 1).reshape(_CHUNK, _TOP_K))
    if _N_CHUNK == 1:
        probs, wts, idxs = probs_cs[0], wts_cs[0], idx_cs[0]
    else:
        probs = jnp.concatenate(probs_cs, 0)
        wts = jnp.concatenate(wts_cs, 0)
        idxs = jnp.concatenate(idx_cs, 0)
    return (
        wts.reshape(b, seq, _TOP_K),
        idxs.reshape(b, seq, _TOP_K),
        probs.reshape(b, seq, _N_EXPERTS),
    )


# speedup vs baseline: 1.1603x; 1.1603x over previous
import functools

import jax
import jax.numpy as jnp
from jax import lax
from jax.experimental import pallas as pl
from jax.experimental.pallas import tpu as pltpu
from jax.experimental.pallas import tpu_sc as plsc

_D_MODEL = 4096
_N_EXPERTS = 80
_TOP_K = 8
_N_TOK = 8192
_N_CHUNK = 1
_CHUNK = _N_TOK // _N_CHUNK
_N_WORKERS = 32
_TPT = _CHUNK // _N_WORKERS
_TB = 256
_WPB = _TB // _TPT
_BPB = (_N_TOK // 2) // _TB

_NEG_INF = float("-inf")


def _dense_body(x_ref, w_ref, b_ref, probs_ref, logt_ref):
    xb = x_ref[...]
    w = w_ref[...]
    logits = lax.dot_general(
        xb, w, dimension_numbers=(((1,), (1,)), ((), ())),
        preferred_element_type=jnp.float32,
    )
    logits = logits + b_ref[...]

    m = jnp.max(logits, axis=1, keepdims=True)
    e = jnp.exp(logits - m)
    probs_ref[...] = e / jnp.sum(e, axis=1, keepdims=True)

    lt = logits.T
    logt_ref[...] = jnp.concatenate(
        [lt[:, w * _TPT:(w + 1) * _TPT].reshape(1, _N_EXPERTS, _TPT)
         for w in range(_WPB)], axis=0)


def _run_dense(x2d, W, bias, chunk):
    grid = (_CHUNK // _TB,)
    base = chunk * (_CHUNK // _TB)
    return pl.pallas_call(
        _dense_body,
        grid=grid,
        in_specs=[
            pl.BlockSpec((_TB, _D_MODEL), lambda i: (base + i, 0)),
            pl.BlockSpec((_N_EXPERTS, _D_MODEL), lambda i: (0, 0)),
            pl.BlockSpec((1, _N_EXPERTS), lambda i: (0, 0)),
        ],
        out_specs=[
            pl.BlockSpec((_TB, _N_EXPERTS), lambda i: (i, 0)),
            pl.BlockSpec((_WPB, _N_EXPERTS, _TPT), lambda i: (i, 0, 0)),
        ],
        out_shape=[
            jax.ShapeDtypeStruct((_CHUNK, _N_EXPERTS), jnp.float32),
            jax.ShapeDtypeStruct((_N_WORKERS, _N_EXPERTS, _TPT), jnp.float32),
        ],
        name=f"router_dense_c{chunk}",
    )(x2d, W, bias)


def _topk_body(logt_hbm, wts_hbm, idx_hbm, buf, wv, iv):
    c = lax.axis_index("c")
    s = lax.axis_index("s")
    wid = s * 2 + c

    pltpu.sync_copy(logt_hbm.at[wid], buf)

    def group(g, carry):
        base = g * 16
        t_v = [jnp.full((16,), _NEG_INF, jnp.float32) for _ in range(_TOP_K)]
        t_i = [jnp.zeros((16,), jnp.int32) for _ in range(_TOP_K)]
        for e in range(_N_EXPERTS):
            v = buf[e, pl.ds(base, 16)]
            ei = jnp.full((16,), e, jnp.int32)
            b = [v > t_v[j] for j in range(_TOP_K)]
            nv = [jnp.where(b[0], v, t_v[0])]
            ni = [jnp.where(b[0], ei, t_i[0])]
            for j in range(1, _TOP_K):
                cv = jnp.where(b[j - 1], t_v[j - 1], v)
                ci = jnp.where(b[j - 1], t_i[j - 1], ei)
                nv.append(jnp.where(b[j], cv, t_v[j]))
                ni.append(jnp.where(b[j], ci, t_i[j]))
            t_v, t_i = nv, ni
        es = [jnp.exp(t_v[j] - t_v[0]) for j in range(_TOP_K)]
        tot = es[0]
        for j in range(1, _TOP_K):
            tot = tot + es[j]
        for j in range(_TOP_K):
            wv[j, pl.ds(base, 16)] = es[j] / tot
            iv[j, pl.ds(base, 16)] = t_i[j]
        return carry

    lax.fori_loop(0, _TPT // 16, group, 0)

    pltpu.sync_copy(wv, wts_hbm.at[wid])
    pltpu.sync_copy(iv, idx_hbm.at[wid])


def _run_topk(logt):
    mesh = plsc.VectorSubcoreMesh(core_axis_name="c", subcore_axis_name="s")
    f = pl.kernel(
        _topk_body,
        out_type=[
            jax.ShapeDtypeStruct((_N_WORKERS, _TOP_K, _TPT), jnp.float32),
            jax.ShapeDtypeStruct((_N_WORKERS, _TOP_K, _TPT), jnp.int32),
        ],
        mesh=mesh,
        scratch_types=[
            pltpu.VMEM((_N_EXPERTS, _TPT), jnp.float32),
            pltpu.VMEM((_TOP_K, _TPT), jnp.float32),
            pltpu.VMEM((_TOP_K, _TPT), jnp.int32),
        ],
    )
    return f(logt)


@jax.jit
def kernel(x, W, expert_bias):
    b, seq, d = x.shape
    x2d = x.reshape(b * seq, d)
    bias2d = expert_bias.reshape(1, _N_EXPERTS)
    probs_cs, wts_cs, idx_cs = [], [], []
    for c in range(_N_CHUNK):
        probs_c, logt_c = _run_dense(x2d, W, bias2d, c)
        wts3, idx3 = _run_topk(logt_c)
        probs_cs.append(probs_c)
        wts_cs.append(wts3.transpose(0, 2, 1).reshape(_CHUNK, _TOP_K))
        idx_cs.append(idx3.transpose(0, 2, 1).reshape(_CHUNK, _TOP_K))
    if _N_CHUNK == 1:
        probs, wts, idxs = probs_cs[0], wts_cs[0], idx_cs[0]
    else:
        probs = jnp.concatenate(probs_cs, 0)
        wts = jnp.concatenate(wts_cs, 0)
        idxs = jnp.concatenate(idx_cs, 0)
    return (
        wts.reshape(b, seq, _TOP_K),
        idxs.reshape(b, seq, _TOP_K),
        probs.reshape(b, seq, _N_EXPERTS),
    )
